# baseline (speedup 1.0000x reference)
import functools

import jax
import jax.numpy as jnp
from jax.experimental import pallas as pl
from jax.experimental.pallas import tpu as pltpu

_EMB = 32
_COMMIT = 0.25


def _vq_tile_kernel(x_ref, w_ref, q_ref, enc_ref, loss_ref, ent_ref,
                    counts_scr, sse_scr, wb_scr, w2_scr, whi_scr, wlo_scr,
                    *, n_tok, num_codes, code_chunk):
    i = pl.program_id(0)
    n = pl.num_programs(0)

    @pl.when(i == 0)
    def _init():
        w = w_ref[...]
        wb_scr[...] = w.astype(jnp.bfloat16) * jnp.bfloat16(2.0)
        w2_scr[...] = jnp.sum(w * w, axis=1)[None, :]
        w_hi = w.astype(jnp.bfloat16)
        whi_scr[...] = w_hi
        wlo_scr[...] = (w - w_hi.astype(jnp.float32)).astype(jnp.bfloat16)
        counts_scr[...] = jnp.zeros_like(counts_scr)
        sse_scr[0] = 0.0

    x = x_ref[...]
    t = x.shape[0]
    x2 = jnp.sum(x * x, axis=1, keepdims=True)
    xb = x.astype(jnp.bfloat16)

    half = num_codes // 2
    lanes = 128
    lane_iota = jax.lax.broadcasted_iota(jnp.int32, (t, lanes), 1)
    big_i = jnp.int32(num_codes)
    bests, bargs = [], []
    for h in range(2):
        bv = jnp.full((t, lanes), jnp.inf, dtype=jnp.float32)
        bi = jnp.zeros((t, lanes), dtype=jnp.int32)
        for cc in range(half // code_chunk):
            c = h * (half // code_chunk) + cc
            wc2 = wb_scr[pl.ds(c * code_chunk, code_chunk), :]
            mm2 = jnp.dot(xb, wc2.T,
                          preferred_element_type=jnp.float32)
            for j in range(code_chunk // lanes):
                base = c * code_chunk + j * lanes
                w2j = w2_scr[0, pl.ds(base, lanes)]
                dj = (x2 + w2j[None, :]) - mm2[:, j * lanes:(j + 1) * lanes]
                upd = dj < bv
                bi = jnp.where(upd, lane_iota + base, bi)
                bv = jnp.where(upd, dj, bv)
        m = jnp.min(bv, axis=1)
        cand = bv == m[:, None]
        a = jnp.min(jnp.where(cand, bi, big_i), axis=1)
        bests.append(m)
        bargs.append(a)
    m0b = bests[0].astype(jnp.bfloat16).astype(jnp.float32)
    idx = jnp.where(bests[1] < m0b, bargs[1], bargs[0])

    iota = jax.lax.broadcasted_iota(jnp.int32, (t, num_codes), 1)
    enc = (iota == idx[:, None]).astype(jnp.float32)
    enc_ref[...] = enc

    encb = enc.astype(jnp.bfloat16)
    q = (jnp.dot(encb, whi_scr[...], preferred_element_type=jnp.float32)
         + jnp.dot(encb, wlo_scr[...], preferred_element_type=jnp.float32))
    q_ref[...] = q

    ones_row = jnp.ones((1, t), dtype=jnp.bfloat16)
    counts_scr[...] += jnp.dot(ones_row, encb,
                               preferred_element_type=jnp.float32)
    diff = q - x
    sse_scr[0] += jnp.sum(diff * diff)

    @pl.when(i == n - 1)
    def _fini():
        loss = _COMMIT * sse_scr[0] / (n_tok * _EMB)
        loss_ref[...] = loss[None, None]
        p = counts_scr[...] / n_tok
        ent = -jnp.sum(p * jnp.log(p + 1e-10))
        ent_ref[...] = ent[None, None]


def kernel(inputs, W):
    b, c, h, w = inputs.shape
    num_codes, emb = W.shape
    x = jnp.transpose(inputs, (0, 2, 3, 1)).reshape(-1, emb)
    n_tok = x.shape[0]

    tok_tile = 64
    code_chunk = 1024
    grid = (n_tok // tok_tile,)

    body = functools.partial(_vq_tile_kernel, n_tok=n_tok,
                             num_codes=num_codes, code_chunk=code_chunk)

    q, enc, loss, ent = pl.pallas_call(
        body,
        grid=grid,
        in_specs=[
            pl.BlockSpec((tok_tile, emb), lambda i: (i, 0)),
            pl.BlockSpec((num_codes, emb), lambda i: (0, 0)),
        ],
        out_specs=[
            pl.BlockSpec((tok_tile, emb), lambda i: (i, 0)),
            pl.BlockSpec((tok_tile, num_codes), lambda i: (i, 0)),
            pl.BlockSpec((1, 1), lambda i: (0, 0)),
            pl.BlockSpec((1, 1), lambda i: (0, 0)),
        ],
        out_shape=[
            jax.ShapeDtypeStruct((n_tok, emb), jnp.float32),
            jax.ShapeDtypeStruct((n_tok, num_codes), jnp.float32),
            jax.ShapeDtypeStruct((1, 1), jnp.float32),
            jax.ShapeDtypeStruct((1, 1), jnp.float32),
        ],
        scratch_shapes=[
            pltpu.VMEM((1, num_codes), jnp.float32),
            pltpu.SMEM((1,), jnp.float32),
            pltpu.VMEM((num_codes, emb), jnp.bfloat16),
            pltpu.VMEM((1, num_codes), jnp.float32),
            pltpu.VMEM((num_codes, emb), jnp.bfloat16),
            pltpu.VMEM((num_codes, emb), jnp.bfloat16),
        ],
    )(x, W)

    out = jnp.transpose(q.reshape(b, h, w, c), (0, 3, 1, 2))
    return out, loss[0, 0], ent[0, 0], enc


# speedup vs baseline: 1.4431x; 1.4431x over previous
import functools

import jax
import jax.numpy as jnp
from jax.experimental import pallas as pl
from jax.experimental.pallas import tpu as pltpu

_EMB = 32
_COMMIT = 0.25


def _vq_tile_kernel(x_ref, w_ref, q_ref, enc_ref, loss_ref, ent_ref,
                    counts_scr, sse_scr, wb_scr, w2_scr, whi_scr, wlo_scr,
                    *, n_tok, num_codes, code_chunk):
    i = pl.program_id(0)
    n = pl.num_programs(0)

    @pl.when(i == 0)
    def _init():
        w = w_ref[...]
        wb_scr[...] = w.astype(jnp.bfloat16) * jnp.bfloat16(2.0)
        w2_scr[...] = jnp.sum(w * w, axis=1)[None, :]
        w_hi = w.astype(jnp.bfloat16)
        whi_scr[...] = w_hi
        wlo_scr[...] = (w - w_hi.astype(jnp.float32)).astype(jnp.bfloat16)
        counts_scr[...] = jnp.zeros_like(counts_scr)
        sse_scr[0] = 0.0

    x = x_ref[...]
    t = x.shape[0]
    x2 = jnp.sum(x * x, axis=1, keepdims=True)
    xb = x.astype(jnp.bfloat16)

    half = num_codes // 2
    lanes = 128
    lane_iota = jax.lax.broadcasted_iota(jnp.int32, (t, lanes), 1)
    big_i = jnp.int32(num_codes)
    bests, bargs = [], []
    for h in range(2):
        bv = jnp.full((t, lanes), jnp.inf, dtype=jnp.float32)
        bi = jnp.zeros((t, lanes), dtype=jnp.int32)
        for cc in range(half // code_chunk):
            c = h * (half // code_chunk) + cc
            wc2 = wb_scr[pl.ds(c * code_chunk, code_chunk), :]
            mm2 = jnp.dot(xb, wc2.T,
                          preferred_element_type=jnp.float32)
            for j in range(code_chunk // lanes):
                base = c * code_chunk + j * lanes
                w2j = w2_scr[0, pl.ds(base, lanes)]
                dj = (x2 + w2j[None, :]) - mm2[:, j * lanes:(j + 1) * lanes]
                upd = dj < bv
                bi = jnp.where(upd, lane_iota + base, bi)
                bv = jnp.where(upd, dj, bv)
        m = jnp.min(bv, axis=1)
        cand = bv == m[:, None]
        a = jnp.min(jnp.where(cand, bi, big_i), axis=1)
        bests.append(m)
        bargs.append(a)
    m0b = bests[0].astype(jnp.bfloat16).astype(jnp.float32)
    idx = jnp.where(bests[1] < m0b, bargs[1], bargs[0])

    iota = jax.lax.broadcasted_iota(jnp.int32, (t, num_codes), 1)
    enc = (iota == idx[:, None]).astype(jnp.float32)
    enc_ref[...] = enc

    encb = enc.astype(jnp.bfloat16)
    q = (jnp.dot(encb, whi_scr[...], preferred_element_type=jnp.float32)
         + jnp.dot(encb, wlo_scr[...], preferred_element_type=jnp.float32))
    q_ref[...] = q

    ones_row = jnp.ones((1, t), dtype=jnp.bfloat16)
    counts_scr[...] += jnp.dot(ones_row, encb,
                               preferred_element_type=jnp.float32)
    diff = q - x
    sse_scr[0] += jnp.sum(diff * diff)

    @pl.when(i == n - 1)
    def _fini():
        loss = _COMMIT * sse_scr[0] / (n_tok * _EMB)
        loss_ref[...] = loss[None, None]
        p = counts_scr[...] / n_tok
        ent = -jnp.sum(p * jnp.log(p + 1e-10))
        ent_ref[...] = ent[None, None]


def kernel(inputs, W):
    b, c, h, w = inputs.shape
    num_codes, emb = W.shape
    x = jnp.transpose(inputs, (0, 2, 3, 1)).reshape(-1, emb)
    n_tok = x.shape[0]

    tok_tile = 256
    code_chunk = 1024
    grid = (n_tok // tok_tile,)

    body = functools.partial(_vq_tile_kernel, n_tok=n_tok,
                             num_codes=num_codes, code_chunk=code_chunk)

    q, enc, loss, ent = pl.pallas_call(
        body,
        grid=grid,
        in_specs=[
            pl.BlockSpec((tok_tile, emb), lambda i: (i, 0)),
            pl.BlockSpec((num_codes, emb), lambda i: (0, 0)),
        ],
        out_specs=[
            pl.BlockSpec((tok_tile, emb), lambda i: (i, 0)),
            pl.BlockSpec((tok_tile, num_codes), lambda i: (i, 0)),
            pl.BlockSpec((1, 1), lambda i: (0, 0)),
            pl.BlockSpec((1, 1), lambda i: (0, 0)),
        ],
        out_shape=[
            jax.ShapeDtypeStruct((n_tok, emb), jnp.float32),
            jax.ShapeDtypeStruct((n_tok, num_codes), jnp.float32),
            jax.ShapeDtypeStruct((1, 1), jnp.float32),
            jax.ShapeDtypeStruct((1, 1), jnp.float32),
        ],
        scratch_shapes=[
            pltpu.VMEM((1, num_codes), jnp.float32),
            pltpu.SMEM((1,), jnp.float32),
            pltpu.VMEM((num_codes, emb), jnp.bfloat16),
            pltpu.VMEM((1, num_codes), jnp.float32),
            pltpu.VMEM((num_codes, emb), jnp.bfloat16),
            pltpu.VMEM((num_codes, emb), jnp.bfloat16),
        ],
    )(x, W)

    out = jnp.transpose(q.reshape(b, h, w, c), (0, 3, 1, 2))
    return out, loss[0, 0], ent[0, 0], enc


# speedup vs baseline: 1.4442x; 1.0008x over previous
import functools

import jax
import jax.numpy as jnp
from jax.experimental import pallas as pl
from jax.experimental.pallas import tpu as pltpu

_EMB = 32
_COMMIT = 0.25


def _vq_tile_kernel(x_ref, w_ref, q_ref, enc_ref, loss_ref, ent_ref,
                    counts_scr, sse_scr, wb_scr, w2_scr, whi_scr, wlo_scr,
                    *, n_tok, num_codes, code_chunk):
    i = pl.program_id(0)
    n = pl.num_programs(0)

    @pl.when(i == 0)
    def _init():
        w = w_ref[...]
        wb_scr[...] = w.astype(jnp.bfloat16) * jnp.bfloat16(2.0)
        w2_scr[...] = jnp.sum(w * w, axis=1)[None, :]
        w_hi = w.astype(jnp.bfloat16)
        whi_scr[...] = w_hi
        wlo_scr[...] = (w - w_hi.astype(jnp.float32)).astype(jnp.bfloat16)
        counts_scr[...] = jnp.zeros_like(counts_scr)
        sse_scr[0] = 0.0

    x = x_ref[...]
    t = x.shape[0]
    x2 = jnp.sum(x * x, axis=1, keepdims=True)
    xb = x.astype(jnp.bfloat16)

    half = num_codes // 2
    lanes = 128
    lane_iota = jax.lax.broadcasted_iota(jnp.int32, (t, lanes), 1)
    big_i = jnp.int32(num_codes)
    bests, bargs = [], []
    for h in range(2):
        bv = jnp.full((t, lanes), jnp.inf, dtype=jnp.float32)
        bi = jnp.zeros((t, lanes), dtype=jnp.int32)
        for cc in range(half // code_chunk):
            c = h * (half // code_chunk) + cc
            wc2 = wb_scr[pl.ds(c * code_chunk, code_chunk), :]
            mm2 = jnp.dot(xb, wc2.T,
                          preferred_element_type=jnp.float32)
            for j in range(code_chunk // lanes):
                base = c * code_chunk + j * lanes
                w2j = w2_scr[0, pl.ds(base, lanes)]
                dj = (x2 + w2j[None, :]) - mm2[:, j * lanes:(j + 1) * lanes]
                upd = dj < bv
                bi = jnp.where(upd, lane_iota + base, bi)
                bv = jnp.where(upd, dj, bv)
        m = jnp.min(bv, axis=1)
        cand = bv == m[:, None]
        a = jnp.min(jnp.where(cand, bi, big_i), axis=1)
        bests.append(m)
        bargs.append(a)
    m0b = bests[0].astype(jnp.bfloat16).astype(jnp.float32)
    idx = jnp.where(bests[1] < m0b, bargs[1], bargs[0])

    iota = jax.lax.broadcasted_iota(jnp.int32, (t, num_codes), 1)
    enc = (iota == idx[:, None]).astype(jnp.float32)
    enc_ref[...] = enc

    encb = enc.astype(jnp.bfloat16)
    q = (jnp.dot(encb, whi_scr[...], preferred_element_type=jnp.float32)
         + jnp.dot(encb, wlo_scr[...], preferred_element_type=jnp.float32))
    q_ref[...] = q

    ones_row = jnp.ones((1, t), dtype=jnp.bfloat16)
    counts_scr[...] += jnp.dot(ones_row, encb,
                               preferred_element_type=jnp.float32)
    diff = q - x
    sse_scr[0] += jnp.sum(diff * diff)

    @pl.when(i == n - 1)
    def _fini():
        loss = _COMMIT * sse_scr[0] / (n_tok * _EMB)
        loss_ref[...] = loss[None, None]
        p = counts_scr[...] / n_tok
        ent = -jnp.sum(p * jnp.log(p + 1e-10))
        ent_ref[...] = ent[None, None]


def kernel(inputs, W):
    b, c, h, w = inputs.shape
    num_codes, emb = W.shape
    x = jnp.transpose(inputs, (0, 2, 3, 1)).reshape(-1, emb)
    n_tok = x.shape[0]

    tok_tile = 256
    code_chunk = 2048
    grid = (n_tok // tok_tile,)

    body = functools.partial(_vq_tile_kernel, n_tok=n_tok,
                             num_codes=num_codes, code_chunk=code_chunk)

    q, enc, loss, ent = pl.pallas_call(
        body,
        grid=grid,
        in_specs=[
            pl.BlockSpec((tok_tile, emb), lambda i: (i, 0)),
            pl.BlockSpec((num_codes, emb), lambda i: (0, 0)),
        ],
        out_specs=[
            pl.BlockSpec((tok_tile, emb), lambda i: (i, 0)),
            pl.BlockSpec((tok_tile, num_codes), lambda i: (i, 0)),
            pl.BlockSpec((1, 1), lambda i: (0, 0)),
            pl.BlockSpec((1, 1), lambda i: (0, 0)),
        ],
        out_shape=[
            jax.ShapeDtypeStruct((n_tok, emb), jnp.float32),
            jax.ShapeDtypeStruct((n_tok, num_codes), jnp.float32),
            jax.ShapeDtypeStruct((1, 1), jnp.float32),
            jax.ShapeDtypeStruct((1, 1), jnp.float32),
        ],
        scratch_shapes=[
            pltpu.VMEM((1, num_codes), jnp.float32),
            pltpu.SMEM((1,), jnp.float32),
            pltpu.VMEM((num_codes, emb), jnp.bfloat16),
            pltpu.VMEM((1, num_codes), jnp.float32),
            pltpu.VMEM((num_codes, emb), jnp.bfloat16),
            pltpu.VMEM((num_codes, emb), jnp.bfloat16),
        ],
    )(x, W)

    out = jnp.transpose(q.reshape(b, h, w, c), (0, 3, 1, 2))
    return out, loss[0, 0], ent[0, 0], enc


# speedup vs baseline: 1.7518x; 1.2130x over previous
import functools

import jax
import jax.numpy as jnp
from jax.experimental import pallas as pl
from jax.experimental.pallas import tpu as pltpu

_EMB = 32
_COMMIT = 0.25


def _vq_tile_kernel(x_ref, w_ref, q_ref, enc_ref, loss_ref, ent_ref,
                    counts_scr, sse_scr, wb_scr, w2_scr, whi_scr,
                    *, n_tok, num_codes, code_chunk):
    i = pl.program_id(0)
    n = pl.num_programs(0)

    @pl.when(i == 0)
    def _init():
        w = w_ref[...]
        wb_scr[...] = w.astype(jnp.bfloat16) * jnp.bfloat16(2.0)
        w2_scr[...] = jnp.sum(w * w, axis=1)[None, :]
        whi_scr[...] = w.astype(jnp.bfloat16)
        counts_scr[...] = jnp.zeros_like(counts_scr)
        sse_scr[0] = 0.0

    x = x_ref[...]
    t = x.shape[0]
    x2 = jnp.sum(x * x, axis=1, keepdims=True)
    xb = x.astype(jnp.bfloat16)

    half = num_codes // 2
    lanes = 128
    lane_iota = jax.lax.broadcasted_iota(jnp.int32, (t, lanes), 1)
    big_i = jnp.int32(num_codes)
    bests, bargs = [], []
    for h in range(2):
        bv = jnp.full((t, lanes), jnp.inf, dtype=jnp.float32)
        bi = jnp.zeros((t, lanes), dtype=jnp.int32)
        for cc in range(half // code_chunk):
            c = h * (half // code_chunk) + cc
            wc2 = wb_scr[pl.ds(c * code_chunk, code_chunk), :]
            mm2 = jnp.dot(xb, wc2.T,
                          preferred_element_type=jnp.float32)
            for j in range(code_chunk // lanes):
                base = c * code_chunk + j * lanes
                w2j = w2_scr[0, pl.ds(base, lanes)]
                dj = (x2 + w2j[None, :]) - mm2[:, j * lanes:(j + 1) * lanes]
                upd = dj < bv
                bi = jnp.where(upd, lane_iota + base, bi)
                bv = jnp.where(upd, dj, bv)
        m = jnp.min(bv, axis=1)
        cand = bv == m[:, None]
        a = jnp.min(jnp.where(cand, bi, big_i), axis=1)
        bests.append(m)
        bargs.append(a)
    m0b = bests[0].astype(jnp.bfloat16).astype(jnp.float32)
    idx = jnp.where(bests[1] < m0b, bargs[1], bargs[0])

    iota = jax.lax.broadcasted_iota(jnp.int32, (t, num_codes), 1)
    enc = (iota == idx[:, None]).astype(jnp.float32)
    enc_ref[...] = enc

    encb = enc.astype(jnp.bfloat16)
    q = jnp.dot(encb, whi_scr[...], preferred_element_type=jnp.float32)
    q_ref[...] = q

    ones_row = jnp.ones((1, t), dtype=jnp.bfloat16)
    counts_scr[...] += jnp.dot(ones_row, encb,
                               preferred_element_type=jnp.float32)
    diff = q - x
    sse_scr[0] += jnp.sum(diff * diff)

    @pl.when(i == n - 1)
    def _fini():
        loss = _COMMIT * sse_scr[0] / (n_tok * _EMB)
        loss_ref[...] = loss[None, None]
        p = counts_scr[...] / n_tok
        ent = -jnp.sum(p * jnp.log(p + 1e-10))
        ent_ref[...] = ent[None, None]


def kernel(inputs, W):
    b, c, h, w = inputs.shape
    num_codes, emb = W.shape
    x = jnp.transpose(inputs, (0, 2, 3, 1)).reshape(-1, emb)
    n_tok = x.shape[0]

    tok_tile = 256
    code_chunk = 2048
    grid = (n_tok // tok_tile,)

    body = functools.partial(_vq_tile_kernel, n_tok=n_tok,
                             num_codes=num_codes, code_chunk=code_chunk)

    q, enc, loss, ent = pl.pallas_call(
        body,
        grid=grid,
        in_specs=[
            pl.BlockSpec((tok_tile, emb), lambda i: (i, 0)),
            pl.BlockSpec((num_codes, emb), lambda i: (0, 0)),
        ],
        out_specs=[
            pl.BlockSpec((tok_tile, emb), lambda i: (i, 0)),
            pl.BlockSpec((tok_tile, num_codes), lambda i: (i, 0)),
            pl.BlockSpec((1, 1), lambda i: (0, 0)),
            pl.BlockSpec((1, 1), lambda i: (0, 0)),
        ],
        out_shape=[
            jax.ShapeDtypeStruct((n_tok, emb), jnp.float32),
            jax.ShapeDtypeStruct((n_tok, num_codes), jnp.float32),
            jax.ShapeDtypeStruct((1, 1), jnp.float32),
            jax.ShapeDtypeStruct((1, 1), jnp.float32),
        ],
        scratch_shapes=[
            pltpu.VMEM((1, num_codes), jnp.float32),
            pltpu.SMEM((1,), jnp.float32),
            pltpu.VMEM((num_codes, emb), jnp.bfloat16),
            pltpu.VMEM((1, num_codes), jnp.float32),
            pltpu.VMEM((num_codes, emb), jnp.bfloat16),
        ],
    )(x, W)

    out = jnp.transpose(q.reshape(b, h, w, c), (0, 3, 1, 2))
    return out, loss[0, 0], ent[0, 0], enc
